# baseline (speedup 1.0000x reference)
import functools

import jax
import jax.numpy as jnp
from jax import lax
from jax.experimental import pallas as pl
from jax.experimental.pallas import tpu as pltpu
from jax.experimental.pallas import tpu_sc as plsc

N = 10000
E = 160000
D = 256
DE = 16
H = 256

NC = 2
NS = 16
N_PAD = 10240
ECH = E // 128
CPC = ECH // NC
NSEG = 5
SEG = CPC // NSEG


def _sc_scatter_kernel(idx_hbm, attr_hbm, out_hbm,
                       idx_v0, idx_v1, val_v0, val_v1, acc, sems):
    c = lax.axis_index("c")
    f = lax.axis_index("s")
    rb = f // 8
    rf = f % 8

    @plsc.parallel_loop(0, N_PAD // 16, unroll=8)
    def _zero(i):
        acc[pl.ds(i * 16, 16)] = jnp.zeros((16,), jnp.float32)

    idx_bufs = (idx_v0, idx_v1)
    val_bufs = (val_v0, val_v1)

    def _start(g):
        cb0 = c * CPC + g * SEG
        di = pltpu.async_copy(idx_hbm.at[pl.ds(cb0, SEG), 0],
                              idx_bufs[g % 2], sems.at[g % 2, 0])
        dv = pltpu.async_copy(attr_hbm.at[rb, pl.ds(cb0, SEG), rf],
                              val_bufs[g % 2], sems.at[g % 2, 1])
        return di, dv

    descs = [None] * NSEG
    descs[0] = _start(0)
    for g in range(NSEG):
        if g + 1 < NSEG:
            descs[g + 1] = _start(g + 1)
        for dd in descs[g]:
            dd.wait()
        idx_v = idx_bufs[g % 2]
        val_v = val_bufs[g % 2]

        @plsc.parallel_loop(0, SEG * 8, unroll=16)
        def _accum(i):
            ch = i // 8
            l = i % 8
            idx16 = idx_v[ch, pl.ds(l * 16, 16)]
            v16 = val_v[ch, pl.ds(l * 16, 16)]
            plsc.addupdate_scatter(acc, [idx16], v16)

    pltpu.sync_copy(acc, out_hbm.at[pl.ds((c * NS + f) * N_PAD, N_PAD)])


def _sc_scatter(idx3, attr4):
    mesh = plsc.VectorSubcoreMesh(core_axis_name="c", subcore_axis_name="s")
    return pl.kernel(
        _sc_scatter_kernel,
        out_type=jax.ShapeDtypeStruct((NC * NS * N_PAD,), jnp.float32),
        mesh=mesh,
        scratch_types=[
            pltpu.VMEM((SEG, 128), jnp.int32),
            pltpu.VMEM((SEG, 128), jnp.int32),
            pltpu.VMEM((SEG, 128), jnp.float32),
            pltpu.VMEM((SEG, 128), jnp.float32),
            pltpu.VMEM((N_PAD,), jnp.float32),
            pltpu.SemaphoreType.DMA((2, 2)),
        ],
        compiler_params=pltpu.CompilerParams(use_tc_tiling_on_sc=False,
                                             needs_layout_passes=False),
    )(idx3, attr4)


BLK = 2048
QCH = BLK // 128


def _mlp_kernel(x_ref, ea_ref, w1x_ref, w1e_ref, b1_ref,
                w2_ref, b2_ref, w3_ref, b3_ref, out_ref):
    hea = jnp.concatenate(
        [jnp.dot((ea_ref[0, :, q, :] + ea_ref[1, :, q, :]).T, w1e_ref[...],
                 preferred_element_type=jnp.float32)
         for q in range(QCH)], axis=0)
    bf = jnp.bfloat16
    h = (jnp.dot(x_ref[...].astype(bf), w1x_ref[...].astype(bf),
                 preferred_element_type=jnp.float32)
         + hea + b1_ref[...])
    h = jnp.where(h >= 0, h, 0.01 * h)
    h = jnp.dot(h.astype(bf), w2_ref[...].astype(bf),
                preferred_element_type=jnp.float32) + b2_ref[...]
    h = jnp.where(h >= 0, h, 0.01 * h)
    h = jnp.dot(h.astype(bf), w3_ref[...].astype(bf),
                preferred_element_type=jnp.float32) + b3_ref[...]
    out_ref[...] = h


def _mlp(x, ea4, w1x, w1e, b1, w2, b2, w3, b3):
    grid = (-(-N // BLK),)
    full = lambda shape: pl.BlockSpec(shape, lambda i: (0,) * len(shape))
    return pl.pallas_call(
        _mlp_kernel,
        grid=grid,
        in_specs=[
            pl.BlockSpec((BLK, D), lambda i: (i, 0)),
            pl.BlockSpec((NC, NS, QCH, 128), lambda i: (0, 0, i, 0)),
            full((D, H)),
            full((DE, H)),
            full((1, H)),
            full((H, H)),
            full((1, H)),
            full((H, H)),
            full((1, H)),
        ],
        out_specs=pl.BlockSpec((BLK, H), lambda i: (i, 0)),
        out_shape=jax.ShapeDtypeStruct((N, H), jnp.float32),
    )(x, ea4, w1x, w1e, b1, w2, b2, w3, b3)


def kernel(x, edge_index, edge_attr, u, batch, W1, b1, W2, b2, W3, b3):
    idx3 = edge_index.reshape(2, ECH, 128).transpose(1, 0, 2)
    attr4 = edge_attr.T.reshape(2, 8, ECH, 128).transpose(0, 2, 1, 3)

    ea_flat = _sc_scatter(idx3, attr4)
    ea4 = ea_flat.reshape(NC, NS, N_PAD // 128, 128)

    return _mlp(x, ea4, W1[:D], W1[D:], b1[None, :],
                W2, b2[None, :], W3, b3[None, :])


# speedup vs baseline: 1.0254x; 1.0254x over previous
import functools

import jax
import jax.numpy as jnp
from jax import lax
from jax.experimental import pallas as pl
from jax.experimental.pallas import tpu as pltpu
from jax.experimental.pallas import tpu_sc as plsc

N = 10000
E = 160000
D = 256
DE = 16
H = 256

NC = 2
NS = 16
N_PAD = 10240
ECH = E // 128
CPC = ECH // NC
NSEG = 5
SEG = CPC // NSEG


def _sc_scatter_kernel(idx_hbm, attr_hbm, out_hbm,
                       idx_v0, idx_v1, val_v0, val_v1, acc, sems):
    c = lax.axis_index("c")
    f = lax.axis_index("s")
    rb = f // 8
    rf = f % 8

    idx_bufs = (idx_v0, idx_v1)
    val_bufs = (val_v0, val_v1)

    def _start(g):
        cb0 = c * CPC + g * SEG
        di = pltpu.async_copy(idx_hbm.at[pl.ds(cb0, SEG), 0],
                              idx_bufs[g % 2], sems.at[g % 2, 0])
        dv = pltpu.async_copy(attr_hbm.at[rb, pl.ds(cb0, SEG), rf],
                              val_bufs[g % 2], sems.at[g % 2, 1])
        return di, dv

    descs = [None] * NSEG
    descs[0] = _start(0)

    @plsc.parallel_loop(0, N_PAD // 16, unroll=8)
    def _zero(i):
        acc[pl.ds(i * 16, 16)] = jnp.zeros((16,), jnp.float32)
    for g in range(NSEG):
        if g + 1 < NSEG:
            descs[g + 1] = _start(g + 1)
        for dd in descs[g]:
            dd.wait()
        idx_v = idx_bufs[g % 2]
        val_v = val_bufs[g % 2]

        @plsc.parallel_loop(0, SEG * 8, unroll=16)
        def _accum(i):
            ch = i // 8
            l = i % 8
            idx16 = idx_v[ch, pl.ds(l * 16, 16)]
            v16 = val_v[ch, pl.ds(l * 16, 16)]
            plsc.addupdate_scatter(acc, [idx16], v16)

    pltpu.sync_copy(acc, out_hbm.at[pl.ds((c * NS + f) * N_PAD, N_PAD)])


def _sc_scatter(idx3, attr4):
    mesh = plsc.VectorSubcoreMesh(core_axis_name="c", subcore_axis_name="s")
    return pl.kernel(
        _sc_scatter_kernel,
        out_type=jax.ShapeDtypeStruct((NC * NS * N_PAD,), jnp.float32),
        mesh=mesh,
        scratch_types=[
            pltpu.VMEM((SEG, 128), jnp.int32),
            pltpu.VMEM((SEG, 128), jnp.int32),
            pltpu.VMEM((SEG, 128), jnp.float32),
            pltpu.VMEM((SEG, 128), jnp.float32),
            pltpu.VMEM((N_PAD,), jnp.float32),
            pltpu.SemaphoreType.DMA((2, 2)),
        ],
        compiler_params=pltpu.CompilerParams(use_tc_tiling_on_sc=False,
                                             needs_layout_passes=False),
    )(idx3, attr4)


BLK = 2048
QCH = BLK // 128


def _mlp_kernel(x_ref, ea_ref, w1x_ref, w1e_ref, b1_ref,
                w2_ref, b2_ref, w3_ref, b3_ref, out_ref):
    hea = jnp.concatenate(
        [jnp.dot((ea_ref[0, :, q, :] + ea_ref[1, :, q, :]).T, w1e_ref[...],
                 preferred_element_type=jnp.float32)
         for q in range(QCH)], axis=0)
    h = (jnp.dot(x_ref[...], w1x_ref[...], preferred_element_type=jnp.float32)
         + hea + b1_ref[...])
    h = jnp.where(h >= 0, h, 0.01 * h)
    h = jnp.dot(h, w2_ref[...], preferred_element_type=jnp.float32) + b2_ref[...]
    h = jnp.where(h >= 0, h, 0.01 * h)
    h = jnp.dot(h, w3_ref[...], preferred_element_type=jnp.float32) + b3_ref[...]
    out_ref[...] = h


def _mlp(x, ea4, w1x, w1e, b1, w2, b2, w3, b3):
    grid = (-(-N // BLK),)
    full = lambda shape: pl.BlockSpec(shape, lambda i: (0,) * len(shape))
    return pl.pallas_call(
        _mlp_kernel,
        grid=grid,
        in_specs=[
            pl.BlockSpec((BLK, D), lambda i: (i, 0)),
            pl.BlockSpec((NC, NS, QCH, 128), lambda i: (0, 0, i, 0)),
            full((D, H)),
            full((DE, H)),
            full((1, H)),
            full((H, H)),
            full((1, H)),
            full((H, H)),
            full((1, H)),
        ],
        out_specs=pl.BlockSpec((BLK, H), lambda i: (i, 0)),
        out_shape=jax.ShapeDtypeStruct((N, H), jnp.float32),
    )(x, ea4, w1x, w1e, b1, w2, b2, w3, b3)


def kernel(x, edge_index, edge_attr, u, batch, W1, b1, W2, b2, W3, b3):
    idx3 = edge_index.reshape(2, ECH, 128).transpose(1, 0, 2)
    attr4 = edge_attr.T.reshape(2, 8, ECH, 128).transpose(0, 2, 1, 3)

    ea_flat = _sc_scatter(idx3, attr4)
    ea4 = ea_flat.reshape(NC, NS, N_PAD // 128, 128)

    return _mlp(x, ea4, W1[:D], W1[D:], b1[None, :],
                W2, b2[None, :], W3, b3[None, :])
